# Initial kernel scaffold; baseline (speedup 1.0000x reference)
import jax
import jax.numpy as jnp
from jax.experimental import pallas as pl


def kernel(input_ids, emb, s_ln_w, s_ln_b, s_Win, s_Win_b, s_conv_w, s_conv_b, s_Wbcdt, s_Wbcdt_b, s_dt_bias, s_Alog, s_D, s_Wout, s_Wout_b, m_ln_w, m_ln_b, m_Wr, m_W1, m_b1, m_W2, m_b2, f_ln_w, f_ln_b):
    raise NotImplementedError("write your pallas kernel here")


# speedup vs baseline: 12.6019x; 12.6019x over previous
import functools

import jax
import jax.numpy as jnp
from jax import lax
from jax.experimental import pallas as pl
from jax.experimental.pallas import tpu as pltpu
from jax.experimental.pallas import tpu_sc as plsc

V = 32000
S = 2048
D = 768
DI = 1536
DS = 16
DC = 4
E = 8
K = 2
DFF = 2048
CAP = 640

EPAD = 648
HPAD = S + 8
NSLOT = E * CAP
SENT = NSLOT

NC = 2
NS = 16
NW = NC * NS

_f32 = jnp.float32
_bf16 = jnp.bfloat16
_i32 = jnp.int32


def _ln_rows(x, w, b):
    m = jnp.mean(x, axis=-1, keepdims=True)
    v = jnp.mean((x - m) ** 2, axis=-1, keepdims=True)
    return (x - m) * lax.rsqrt(v + 1e-5) * w + b


def _sc_mesh():
    return plsc.VectorSubcoreMesh(core_axis_name="c", subcore_axis_name="s")


def _sc_gather_rows(table, idx, n_rows, d, per_chunk):
    n_per_w = n_rows // NW
    n_chunks = n_per_w // per_chunk
    assert n_per_w % per_chunk == 0 and per_chunk <= 128 and per_chunk % 8 == 0

    @functools.partial(
        pl.kernel,
        mesh=_sc_mesh(),
        out_type=jax.ShapeDtypeStruct((n_rows, d), _f32),
        scratch_types=[
            [pltpu.VMEM((per_chunk,), _i32) for _ in range(n_chunks)],
            [pltpu.VMEM((per_chunk, d), _f32) for _ in range(n_chunks)],
            [pltpu.SemaphoreType.DMA for _ in range(n_chunks)],
        ],
    )
    def k(table_hbm, idx_hbm, out_hbm, idx_vs, row_vs, sems):
        wid = lax.axis_index("s") * NC + lax.axis_index("c")
        base = wid * n_per_w
        copies = []
        for c in range(n_chunks):
            pltpu.sync_copy(idx_hbm.at[pl.ds(base + c * per_chunk, per_chunk)],
                            idx_vs[c])
            copies.append(
                pltpu.async_copy(table_hbm.at[idx_vs[c]], row_vs[c], sems[c]))
        for c in range(n_chunks):
            copies[c].wait()
            pltpu.sync_copy(
                row_vs[c], out_hbm.at[pl.ds(base + c * per_chunk, per_chunk)])

    return k(table, idx)


_CH = 256
_NCH = S // _CH


def _ssm_body(x_ref, win_ref, winb_ref, convw_ref, convb_ref, wdt_ref,
              dtb_ref, wb_ref, bb_ref, wc_ref, cb_ref, at_ref, sd_ref,
              wout_ref, woutb_ref, lnw_ref, lnb_ref,
              out_ref,
              h_ref, tail_ref, xc_ref, u_ref, ys_ref, da_ref,
              dt_ref, bm_ref, cm_ref):
    pid = pl.program_id(0)

    @pl.when(pid == 0)
    def _():
        h_ref[...] = jnp.zeros((DS, DI), _f32)
        tail_ref[...] = jnp.zeros((8, DI), _f32)

    x = x_ref[...]
    h = _ln_rows(x, lnw_ref[...], lnb_ref[...])
    xz = jnp.dot(h, win_ref[...],
                 preferred_element_type=_f32) + winb_ref[...]
    xs = xz[:, :DI]
    z = xz[:, DI:]

    tail = tail_ref[5:8, :]
    xs_ext = jnp.concatenate([tail, xs], axis=0)
    xc = convb_ref[...]
    for kk in range(DC):
        xc = xc + xs_ext[kk:kk + _CH, :] * convw_ref[kk:kk + 1, :]
    xc = xc * jax.nn.sigmoid(xc)
    tail_ref[5:8, :] = xs[_CH - 3:_CH, :]

    dt_pre = jnp.dot(xc, wdt_ref[...], preferred_element_type=_f32)
    dt_pre = dt_pre + dtb_ref[...]
    dt = jnp.maximum(dt_pre, 0.0) + jnp.log1p(jnp.exp(-jnp.abs(dt_pre)))
    bm = jnp.dot(xc, wb_ref[...], preferred_element_type=_f32) \
        + bb_ref[...]
    cm = jnp.dot(xc, wc_ref[...], preferred_element_type=_f32) \
        + cb_ref[...]

    xc_ref[...] = xc
    u_ref[...] = dt * xc
    dt_ref[...] = dt
    bm_ref[...] = bm
    cm_ref[...] = cm

    at = at_ref[...]
    eye16 = jnp.eye(DS, dtype=_f32)
    hstate = h_ref[...]

    def blk_body(blk, hstate):
        t0 = blk * 16
        dt16 = dt_ref[pl.ds(t0, 16), :]
        b16 = bm_ref[pl.ds(t0, 16), :]
        c16 = cm_ref[pl.ds(t0, 16), :]
        bt = lax.dot_general(b16, eye16, (((0,), (0,)), ((), ())),
                             preferred_element_type=_f32)
        ct = lax.dot_general(c16, eye16, (((0,), (0,)), ((), ())),
                             preferred_element_type=_f32)
        da_ref[...] = jnp.exp(dt16[:, None, :] * at[None, :, :])
        u16 = u_ref[pl.ds(t0, 16), :]
        for tl in range(16):
            da = da_ref[tl]
            hstate = da * hstate + bt[:, tl:tl + 1] * u16[tl:tl + 1, :]
            y = jnp.sum(hstate * ct[:, tl:tl + 1], axis=0, keepdims=True)
            ys_ref[pl.ds(t0 + tl, 1), :] = y
        return hstate

    hstate = lax.fori_loop(0, _CH // 16, blk_body, hstate, unroll=False)
    h_ref[...] = hstate

    y = ys_ref[...] + sd_ref[...] * xc_ref[...]
    y = y * (z * jax.nn.sigmoid(z))
    out = jnp.dot(y, wout_ref[...],
                  preferred_element_type=_f32) + woutb_ref[...]
    out_ref[...] = x + out


def _ssm_block(x, win_bf, winb, convw_t, convb, wdt_bf, dtb, wb_bf, bb,
               wc_bf, cb, a_t, sd, wout_bf, woutb, lnw, lnb):
    const = pl.BlockSpec((None,), lambda i: (0,))
    full2 = lambda shape: pl.BlockSpec(shape, lambda i: (0, 0))
    return pl.pallas_call(
        _ssm_body,
        grid=(_NCH,),
        in_specs=[
            pl.BlockSpec((_CH, D), lambda i: (i, 0)),
            full2((D, 2 * DI)), full2((1, 2 * DI)),
            full2((DC, DI)), full2((1, DI)),
            full2((DI, DI)), full2((1, DI)),
            full2((DI, DS)), full2((1, DS)),
            full2((DI, DS)), full2((1, DS)),
            full2((DS, DI)), full2((1, DI)),
            full2((DI, D)), full2((1, D)),
            full2((1, D)), full2((1, D)),
        ],
        out_specs=pl.BlockSpec((_CH, D), lambda i: (i, 0)),
        out_shape=jax.ShapeDtypeStruct((S, D), _f32),
        scratch_shapes=[
            pltpu.VMEM((DS, DI), _f32),
            pltpu.VMEM((8, DI), _f32),
            pltpu.VMEM((_CH, DI), _f32),
            pltpu.VMEM((_CH, DI), _f32),
            pltpu.VMEM((_CH, DI), _f32),
            pltpu.VMEM((16, DS, DI), _f32),
            pltpu.VMEM((_CH, DI), _f32),
            pltpu.VMEM((_CH, DS), _f32),
            pltpu.VMEM((_CH, DS), _f32),
        ],
    )(x, win_bf, winb, convw_t, convb, wdt_bf, dtb, wb_bf, bb, wc_bf, cb,
      a_t, sd, wout_bf, woutb, lnw, lnb)


def _route_body(x_ref, lnw_ref, lnb_ref, wr_ref,
                hh_ref, src_ref, comb_ref, gate_ref):
    x = x_ref[...]
    hh = _ln_rows(x, lnw_ref[...], lnb_ref[...])
    hh_ref[0:S, :] = hh
    hh_ref[S:HPAD, :] = jnp.zeros((HPAD - S, D), _f32)

    rl = jnp.dot(hh, wr_ref[...], preferred_element_type=_f32)
    ei = lax.broadcasted_iota(_i32, (S, E), 1)
    v1 = jnp.max(rl, axis=1, keepdims=True)
    i1 = jnp.min(jnp.where(rl == v1, ei, E), axis=1, keepdims=True)
    rl2 = jnp.where(ei == i1, -1e30, rl)
    v2 = jnp.max(rl2, axis=1, keepdims=True)
    i2 = jnp.min(jnp.where(rl2 == v2, ei, E), axis=1, keepdims=True)

    g0 = 1.0 / (1.0 + jnp.exp(v2 - v1))
    gate_ref[...] = jnp.concatenate([g0, 1.0 - g0], axis=1)

    oh0 = (ei == i1).astype(_f32)
    oh1 = (ei == i2).astype(_f32)
    ohsum = oh0 + oh1

    cs = 512
    ri = lax.broadcasted_iota(_i32, (cs, cs), 0)
    ci = lax.broadcasted_iota(_i32, (cs, cs), 1)
    tri = (ri > ci).astype(_f32)
    prefs = []
    carry = jnp.zeros((1, E), _f32)
    for c in range(S // cs):
        ohc = ohsum[c * cs:(c + 1) * cs, :]
        prefs.append(jnp.dot(tri, ohc, preferred_element_type=_f32) + carry)
        carry = carry + jnp.sum(ohc, axis=0, keepdims=True)
    pref = jnp.concatenate(prefs, axis=0)

    pos0 = jnp.sum(pref * oh0, axis=1, keepdims=True).astype(_i32)
    pos1 = jnp.sum(pref * oh1, axis=1, keepdims=True).astype(_i32)

    slot0 = jnp.where(pos0 < CAP, i1 * CAP + pos0, SENT)
    slot1 = jnp.where(pos1 < CAP, i2 * CAP + pos1, SENT)

    comb0 = i1 * EPAD + jnp.minimum(pos0, CAP)
    comb1 = i2 * EPAD + jnp.minimum(pos1, CAP)
    comb_ref[...] = jnp.concatenate([comb0, comb1], axis=1)

    tokp1 = lax.broadcasted_iota(_i32, (S, 1), 0) + 1
    scs = 512
    for c in range(NSLOT // scs):
        sids = lax.broadcasted_iota(_i32, (S, scs), 1) + c * scs
        acc = jnp.sum(jnp.where(slot0 == sids, tokp1, 0)
                      + jnp.where(slot1 == sids, tokp1, 0),
                      axis=0, keepdims=True)
        src_ref[:, c * scs:(c + 1) * scs] = jnp.where(acc == 0, S, acc - 1)


def _route(x, lnw, lnb, wr):
    return pl.pallas_call(
        _route_body,
        out_shape=(
            jax.ShapeDtypeStruct((HPAD, D), _f32),
            jax.ShapeDtypeStruct((1, NSLOT), _i32),
            jax.ShapeDtypeStruct((S, K), _i32),
            jax.ShapeDtypeStruct((S, K), _f32),
        ),
    )(x, lnw, lnb, wr)


def _expert_body(buf_ref, w1_ref, b1_ref, w2_ref, b2_ref, out_ref):
    xb = buf_ref[0].astype(_bf16)
    h1 = jnp.dot(xb, w1_ref[0], preferred_element_type=_f32) + b1_ref[0]
    c0 = 0.7978845608028654
    h1g = 0.5 * h1 * (1.0 + jnp.tanh(c0 * (h1 + 0.044715 * h1 * h1 * h1)))
    eo = jnp.dot(h1g.astype(_bf16), w2_ref[0],
                 preferred_element_type=_f32) + b2_ref[0]
    out_ref[0, 0:CAP, :] = eo
    out_ref[0, CAP:EPAD, :] = jnp.zeros((EPAD - CAP, D), _f32)


def _experts(buf, w1_bf, b1, w2_bf, b2):
    return pl.pallas_call(
        _expert_body,
        grid=(E,),
        in_specs=[
            pl.BlockSpec((1, CAP, D), lambda e: (e, 0, 0)),
            pl.BlockSpec((1, D, DFF), lambda e: (e, 0, 0)),
            pl.BlockSpec((1, 1, DFF), lambda e: (e, 0, 0)),
            pl.BlockSpec((1, DFF, D), lambda e: (e, 0, 0)),
            pl.BlockSpec((1, 1, D), lambda e: (e, 0, 0)),
        ],
        out_specs=pl.BlockSpec((1, EPAD, D), lambda e: (e, 0, 0)),
        out_shape=jax.ShapeDtypeStruct((E, EPAD, D), _f32),
    )(buf, w1_bf, b1, w2_bf, b2)


def _combine_body(x_ref, rows_ref, gate_ref, lnw_ref, lnb_ref, out_ref):
    g0 = gate_ref[:, 0:1]
    g1 = gate_ref[:, 1:2]
    x2 = x_ref[...] + g0 * rows_ref[:, 0:D] + g1 * rows_ref[:, D:2 * D]
    out_ref[...] = _ln_rows(x2, lnw_ref[...], lnb_ref[...]).astype(_bf16)


def _combine(x, rows2, gates, lnw, lnb):
    return pl.pallas_call(
        _combine_body,
        out_shape=jax.ShapeDtypeStruct((S, D), _bf16),
    )(x, rows2, gates, lnw, lnb)


_VT = 1280
_NVT = V // _VT


def _lmhead_body(xo_ref, emb_ref, out_ref):
    eb = emb_ref[...].astype(_bf16)
    out_ref[...] = lax.dot_general(
        xo_ref[...], eb, (((1,), (1,)), ((), ())),
        preferred_element_type=_f32)


def _lmhead(xo_bf, emb):
    return pl.pallas_call(
        _lmhead_body,
        grid=(_NVT,),
        in_specs=[
            pl.BlockSpec((S, D), lambda i: (0, 0)),
            pl.BlockSpec((_VT, D), lambda i: (i, 0)),
        ],
        out_specs=pl.BlockSpec((S, _VT), lambda i: (0, i)),
        out_shape=jax.ShapeDtypeStruct((S, V), _f32),
    )(xo_bf, emb)


def kernel(input_ids, emb, s_ln_w, s_ln_b, s_Win, s_Win_b, s_conv_w,
           s_conv_b, s_Wbcdt, s_Wbcdt_b, s_dt_bias, s_Alog, s_D, s_Wout,
           s_Wout_b, m_ln_w, m_ln_b, m_Wr, m_W1, m_b1, m_W2, m_b2,
           f_ln_w, f_ln_b):
    ids = input_ids.reshape(S).astype(_i32)

    win_f = s_Win
    convw_t = s_conv_w.T
    wdt_f = s_Wbcdt[:, :DI]
    wb_f = s_Wbcdt[:, DI:DI + DS]
    wc_f = s_Wbcdt[:, DI + DS:]
    dtb = (s_Wbcdt_b[:DI] + s_dt_bias).reshape(1, DI)
    bb = s_Wbcdt_b[DI:DI + DS].reshape(1, DS)
    cb = s_Wbcdt_b[DI + DS:].reshape(1, DS)
    a_t = (-jnp.exp(s_Alog)).T
    wout_f = s_Wout
    w1_bf = m_W1.astype(_bf16)
    w2_bf = m_W2.astype(_bf16)

    x0 = _sc_gather_rows(emb, ids, S, D, per_chunk=64)

    x1 = _ssm_block(
        x0, win_f, s_Win_b.reshape(1, 2 * DI), convw_t,
        s_conv_b.reshape(1, DI), wdt_f, dtb, wb_f, bb, wc_f, cb, a_t,
        s_D.reshape(1, DI), wout_f, s_Wout_b.reshape(1, D),
        s_ln_w.reshape(1, D), s_ln_b.reshape(1, D))

    hh_pad, src, comb, gates = _route(
        x1, m_ln_w.reshape(1, D), m_ln_b.reshape(1, D), m_Wr)

    buf = _sc_gather_rows(hh_pad, src.reshape(NSLOT), NSLOT, D, per_chunk=80)

    eo = _experts(buf.reshape(E, CAP, D), w1_bf, m_b1.reshape(E, 1, DFF),
                  w2_bf, m_b2.reshape(E, 1, D))

    rows = _sc_gather_rows(eo.reshape(E * EPAD, D), comb.reshape(S * K),
                           S * K, D, per_chunk=128)

    xo_bf = _combine(x1, rows.reshape(S, K * D), gates,
                     f_ln_w.reshape(1, D), f_ln_b.reshape(1, D))

    logits = _lmhead(xo_bf, emb)
    return logits.reshape(1, S, V)
